# baseline (speedup 1.0000x reference)
import functools

import jax
import jax.numpy as jnp
from jax import lax
from jax.experimental import pallas as pl
from jax.experimental.pallas import tpu as pltpu
from jax.experimental.pallas import tpu_sc as plsc

NC = 2
NS = 16
NW = NC * NS
CHUNK = 128


def _round_up(a, b):
  return (a + b - 1) // b * b


def _make_deg_kernel(Np, nch):
  mesh = plsc.VectorSubcoreMesh(core_axis_name="c", subcore_axis_name="s")
  maxcpt = -(-nch // NW)
  tp = Np // NS

  @functools.partial(
      pl.kernel,
      out_type=jax.ShapeDtypeStruct((NC * 2 * Np,), jnp.float32),
      mesh=mesh,
      compiler_params=pltpu.CompilerParams(use_tc_tiling_on_sc=False),
      scratch_types=[
          pltpu.VMEM((maxcpt, 2, CHUNK), jnp.int32),
          pltpu.VMEM((CHUNK,), jnp.float32),
          pltpu.VMEM((tp,), jnp.float32),
          pltpu.VMEM_SHARED((Np,), jnp.float32),
          pltpu.VMEM_SHARED((Np,), jnp.float32),
          pltpu.SemaphoreType.DMA,
      ],
  )
  def deg_kernel(edge_hbm, out_hbm, edge_v, ones_v, stage_v,
                 dega_sh, degb_sh, sem):
    c = lax.axis_index("c")
    s = lax.axis_index("s")
    w = c * NS + s
    cs = (nch * w) // NW
    ce = (nch * (w + 1)) // NW
    t = ce - cs
    pltpu.sync_copy(edge_hbm.at[pl.ds(cs, maxcpt)], edge_v)
    for k in range(CHUNK // 16):
      ones_v[pl.ds(k * 16, 16)] = jnp.full((16,), 1.0, jnp.float32)

    def zbody(j, carry):
      stage_v[pl.ds(j * 16, 16)] = jnp.zeros((16,), jnp.float32)
      return carry

    lax.fori_loop(0, tp // 16, zbody, 0)
    off = pl.multiple_of(s * tp, 8)
    pltpu.sync_copy(stage_v, dega_sh.at[pl.ds(off, tp)])
    pltpu.sync_copy(stage_v, degb_sh.at[pl.ds(off, tp)])
    plsc.subcore_barrier()

    def body(j, carry):
      @pl.when(j < t)
      def _():
        pltpu.async_copy(ones_v, dega_sh.at[edge_v.at[j, 0]], sem, add=True)
        pltpu.async_copy(ones_v, degb_sh.at[edge_v.at[j, 1]], sem,
                         add=True)

      return carry

    lax.fori_loop(0, maxcpt, body, 0)

    def drain(j, carry):
      pltpu.make_async_copy(ones_v, dega_sh.at[edge_v.at[0, 0]], sem).wait()
      return carry

    lax.fori_loop(0, 2 * t, drain, 0)
    plsc.subcore_barrier()
    base = c * 2 * Np
    pltpu.sync_copy(dega_sh.at[pl.ds(off, tp)], stage_v)
    pltpu.sync_copy(stage_v,
                    out_hbm.at[pl.ds(pl.multiple_of(base + s * tp, 8), tp)])
    pltpu.sync_copy(degb_sh.at[pl.ds(off, tp)], stage_v)
    pltpu.sync_copy(stage_v,
                    out_hbm.at[pl.ds(pl.multiple_of(base + Np + s * tp, 8),
                                     tp)])

  return deg_kernel


def _make_prop_kernel(Np, H, nch):
  mesh = plsc.VectorSubcoreMesh(core_axis_name="c", subcore_axis_name="s")
  maxcpt = -(-nch // NW)
  rpt = Np // NS
  nq = rpt // CHUNK
  assert rpt % CHUNK == 0
  NSLOT = 6
  PD = 4

  @functools.partial(
      pl.kernel,
      out_type=jax.ShapeDtypeStruct((NC * NS * nq, 2, CHUNK, H), jnp.float32),
      mesh=mesh,
      compiler_params=pltpu.CompilerParams(use_tc_tiling_on_sc=False),
      scratch_types=[
          pltpu.VMEM((maxcpt, 2, CHUNK), jnp.int32),
          pltpu.VMEM((NSLOT, CHUNK, H), jnp.float32),
          pltpu.VMEM_SHARED((Np, H), jnp.float32),
      ] + [pltpu.SemaphoreType.DMA] * (2 * NSLOT),
  )
  def prop_kernel(table_hbm, edge_hbm, out_hbm,
                  edge_v, rows_v, acc_sh, *sems):
    c = lax.axis_index("c")
    s = lax.axis_index("s")
    w = c * NS + s
    cs = (nch * w) // NW
    ce = (nch * (w + 1)) // NW
    t = ce - cs
    sgs = sems[:NSLOT]
    sss = sems[NSLOT:]
    pltpu.sync_copy(edge_hbm.at[pl.ds(cs, maxcpt)], edge_v)

    def dbl(j, carry):
      for k in range(CHUNK // 16):
        v = edge_v[j, 0, pl.ds(k * 16, 16)]
        edge_v[j, 0, pl.ds(k * 16, 16)] = v + v
      return carry

    lax.fori_loop(0, maxcpt, dbl, 0)

    def zbody(j, carry):
      for k in range(H // 16):
        rows_v[0, j, pl.ds(k * 16, 16)] = jnp.zeros((16,), jnp.float32)
      return carry

    lax.fori_loop(0, CHUNK, zbody, 0)
    for q in range(nq):
      pltpu.sync_copy(rows_v.at[0],
                      acc_sh.at[pl.ds(pl.multiple_of(s * rpt + q * CHUNK, 8),
                                      CHUNK)])
    plsc.subcore_barrier()

    for b in range(PD):
      @pl.when(b < t)
      def _(b=b):
        pltpu.async_copy(table_hbm.at[edge_v.at[b, 0]], rows_v.at[b], sgs[b])

    def body(jo, carry):
      for b in range(NSLOT):
        jj = NSLOT * jo + b
        pj = jj + PD
        pb = (b + PD) % NSLOT

        @pl.when(pj < t)
        def _(jj=jj, pj=pj, pb=pb):
          @pl.when(pj >= NSLOT)
          def _():
            pltpu.make_async_copy(rows_v.at[pb],
                                  acc_sh.at[edge_v.at[pj - NSLOT, 1]],
                                  sss[pb]).wait()

          pltpu.async_copy(table_hbm.at[edge_v.at[pj, 0]], rows_v.at[pb],
                           sgs[pb])

        @pl.when(jj < t)
        def _(jj=jj, b=b):
          pltpu.make_async_copy(table_hbm.at[edge_v.at[jj, 0]], rows_v.at[b],
                                sgs[b]).wait()
          pltpu.async_copy(rows_v.at[b], acc_sh.at[edge_v.at[jj, 1]],
                           sss[b], add=True)

      return carry

    _ = lax.fori_loop(0, (maxcpt + NSLOT - 1) // NSLOT, body, 0)

    for b in range(NSLOT):
      @pl.when(b < t)
      def _(b=b):
        last = ((t - 1 - b) // NSLOT) * NSLOT + b
        pltpu.make_async_copy(rows_v.at[b],
                              acc_sh.at[edge_v.at[last, 1]],
                              sss[b]).wait()

    plsc.subcore_barrier()

    for q in range(3):
      if q < nq:
        p = (q % 3) * 2
        aq = pl.multiple_of(s * rpt + q * CHUNK, 8)
        pltpu.async_copy(acc_sh.at[pl.ds(aq, CHUNK)], rows_v.at[p], sgs[p])
    for q in range(nq):
      p = (q % 3) * 2
      aq = pl.multiple_of(s * rpt + q * CHUNK, 8)
      pltpu.make_async_copy(acc_sh.at[pl.ds(aq, CHUNK)], rows_v.at[p],
                            sgs[p]).wait()

      def spread_hi(i, carry, p=p):
        j = 64 + i
        for k in range(H // 16):
          rows_v[p + 1, 2 * j - 128, pl.ds(k * 16, 16)] = (
              rows_v[p, j, pl.ds(k * 16, 16)])
        return carry

      lax.fori_loop(0, 64, spread_hi, 0)

      def spread_lo(i, carry, p=p):
        j = 63 - i
        for k in range(H // 16):
          rows_v[p, 2 * j, pl.ds(k * 16, 16)] = (
              rows_v[p, j, pl.ds(k * 16, 16)])
        return carry

      lax.fori_loop(0, 63, spread_lo, 0)

      oi = (c * NS + s) * nq + q
      pltpu.async_copy(rows_v.at[pl.ds(p, 2)], out_hbm.at[oi], sss[p])
      if q + 3 < nq:
        pltpu.make_async_copy(rows_v.at[pl.ds(p, 2)], out_hbm.at[oi],
                              sss[p]).wait()
        aq2 = pl.multiple_of(s * rpt + (q + 3) * CHUNK, 8)
        pltpu.async_copy(acc_sh.at[pl.ds(aq2, CHUNK)], rows_v.at[p], sgs[p])
    for q in range(max(nq - 3, 0), nq):
      p = (q % 3) * 2
      oi = (c * NS + s) * nq + q
      pltpu.make_async_copy(rows_v.at[pl.ds(p, 2)], out_hbm.at[oi],
                            sss[p]).wait()

  return prop_kernel


def _norms(deg_ref, Np, blk, i, which):
  d0 = deg_ref[pl.ds(which * Np + i * blk, blk)]
  d1 = deg_ref[pl.ds((2 + which) * Np + i * blk, blk)]
  n = lax.rsqrt(jnp.maximum(d0 + d1, 1.0))
  return jnp.reshape(n, (blk, 1))


def _prep_body(Np, blk, x_ref, w1_ref, deg_ref, g1_ref):
  i = pl.program_id(0)
  nsrc = _norms(deg_ref, Np, blk, i, 0)
  h = jnp.dot(x_ref[...], w1_ref[...], preferred_element_type=jnp.float32)
  g = h * nsrc
  g1_ref[...] = jnp.concatenate([g, jnp.zeros_like(g)], axis=1)


def _mid_body(Np, blk, p_ref, deg_ref, o_ref):
  i = pl.program_id(0)
  nsrc = _norms(deg_ref, Np, blk, i, 0)
  ndst = _norms(deg_ref, Np, blk, i, 1)
  pv = p_ref[...]
  H = pv.shape[-1] // 2
  agg = pv[0, :, :H] + pv[1, :, :H]
  h1 = jnp.maximum(agg * ndst, 0.0)
  h = h1 * nsrc
  o_ref[...] = jnp.concatenate([h, jnp.zeros_like(h)], axis=1)


def _dec_body(Np, blk, q_ref, deg_ref, w2_ref, w3_ref, mu_ref, lv_ref):
  i = pl.program_id(0)
  ndst = _norms(deg_ref, Np, blk, i, 1)
  qv = q_ref[...]
  H = qv.shape[-1] // 2
  sblk = (qv[0, :, :H] + qv[1, :, :H]) * ndst
  mu_ref[...] = jnp.dot(sblk, w2_ref[...], preferred_element_type=jnp.float32)
  lv_ref[...] = jnp.dot(sblk, w3_ref[...], preferred_element_type=jnp.float32)


def _adj_body(a_ref, b_ref, o_ref):
  o_ref[...] = lax.dot_general(a_ref[...], b_ref[...],
                               (((1,), (1,)), ((), ())),
                               preferred_element_type=jnp.float32)


def kernel(x, edge_index, W1, W2, W3):
  N, D = x.shape
  E = edge_index.shape[1]
  H1 = W1.shape[1]
  H2 = W2.shape[1]

  Np = _round_up(N, 2048)
  blk = 2048
  nblk = Np // blk
  nch = E // CHUNK
  assert E % CHUNK == 0

  edge_r = jnp.transpose(edge_index.reshape(2, nch, CHUNK), (1, 0, 2))

  deg_flat = _make_deg_kernel(Np, nch)(edge_r)

  x_pad = jnp.pad(x, ((0, Np - N), (0, 0)))
  g1 = pl.pallas_call(
      functools.partial(_prep_body, Np, blk),
      grid=(nblk,),
      in_specs=[
          pl.BlockSpec((blk, D), lambda i: (i, 0)),
          pl.BlockSpec((D, H1), lambda i: (0, 0)),
          pl.BlockSpec((NC * 2 * Np,), lambda i: (0,)),
      ],
      out_specs=pl.BlockSpec((blk, 2 * H1), lambda i: (i, 0)),
      out_shape=jax.ShapeDtypeStruct((Np, 2 * H1), jnp.float32),
  )(x_pad, W1, deg_flat)
  g1t = g1.reshape(2 * Np, H1)

  prop = _make_prop_kernel(Np, H1, nch)
  agg_parts = prop(g1t, edge_r).reshape(NC, Np, 2 * H1)

  h1n = pl.pallas_call(
      functools.partial(_mid_body, Np, blk),
      grid=(nblk,),
      in_specs=[
          pl.BlockSpec((NC, blk, 2 * H1), lambda i: (0, i, 0)),
          pl.BlockSpec((NC * 2 * Np,), lambda i: (0,)),
      ],
      out_specs=pl.BlockSpec((blk, 2 * H1), lambda i: (i, 0)),
      out_shape=jax.ShapeDtypeStruct((Np, 2 * H1), jnp.float32),
  )(agg_parts, deg_flat)
  h1nt = h1n.reshape(2 * Np, H1)

  q_parts = prop(h1nt, edge_r).reshape(NC, Np, 2 * H1)

  mu, logvar = pl.pallas_call(
      functools.partial(_dec_body, Np, blk),
      grid=(nblk,),
      in_specs=[
          pl.BlockSpec((NC, blk, 2 * H1), lambda i: (0, i, 0)),
          pl.BlockSpec((NC * 2 * Np,), lambda i: (0,)),
          pl.BlockSpec((H1, H2), lambda i: (0, 0)),
          pl.BlockSpec((H1, H2), lambda i: (0, 0)),
      ],
      out_specs=[
          pl.BlockSpec((blk, H2), lambda i: (i, 0)),
          pl.BlockSpec((blk, H2), lambda i: (i, 0)),
      ],
      out_shape=[
          jax.ShapeDtypeStruct((N, H2), jnp.float32),
          jax.ShapeDtypeStruct((N, H2), jnp.float32),
      ],
  )(q_parts, deg_flat, W2, W3)

  bm = 400
  adj = pl.pallas_call(
      _adj_body,
      grid=(N // bm,),
      in_specs=[
          pl.BlockSpec((bm, H2), lambda i: (i, 0)),
          pl.BlockSpec((N, H2), lambda i: (0, 0)),
      ],
      out_specs=pl.BlockSpec((bm, N), lambda i: (i, 0)),
      out_shape=jax.ShapeDtypeStruct((N, N), jnp.float32),
  )(mu, mu)

  return (adj, mu, logvar)


# speedup vs baseline: 1.0082x; 1.0082x over previous
import functools

import jax
import jax.numpy as jnp
from jax import lax
from jax.experimental import pallas as pl
from jax.experimental.pallas import tpu as pltpu
from jax.experimental.pallas import tpu_sc as plsc

NC = 2
NS = 16
NW = NC * NS
CHUNK = 128


def _round_up(a, b):
  return (a + b - 1) // b * b


def _make_deg_kernel(Np, nch):
  mesh = plsc.VectorSubcoreMesh(core_axis_name="c", subcore_axis_name="s")
  maxcpt = -(-nch // NW)
  tp = Np // NS

  @functools.partial(
      pl.kernel,
      out_type=jax.ShapeDtypeStruct((NC * 2 * Np,), jnp.float32),
      mesh=mesh,
      compiler_params=pltpu.CompilerParams(use_tc_tiling_on_sc=False),
      scratch_types=[
          pltpu.VMEM((maxcpt, 2, CHUNK), jnp.int32),
          pltpu.VMEM((CHUNK,), jnp.float32),
          pltpu.VMEM((tp,), jnp.float32),
          pltpu.VMEM_SHARED((Np,), jnp.float32),
          pltpu.VMEM_SHARED((Np,), jnp.float32),
          pltpu.SemaphoreType.DMA,
      ],
  )
  def deg_kernel(edge_hbm, out_hbm, edge_v, ones_v, stage_v,
                 dega_sh, degb_sh, sem):
    c = lax.axis_index("c")
    s = lax.axis_index("s")
    w = c * NS + s
    cs = (nch * w) // NW
    ce = (nch * (w + 1)) // NW
    t = ce - cs
    pltpu.sync_copy(edge_hbm.at[pl.ds(cs, maxcpt)], edge_v)
    for k in range(CHUNK // 16):
      ones_v[pl.ds(k * 16, 16)] = jnp.full((16,), 1.0, jnp.float32)

    def zbody(j, carry):
      stage_v[pl.ds(j * 16, 16)] = jnp.zeros((16,), jnp.float32)
      return carry

    lax.fori_loop(0, tp // 16, zbody, 0)
    off = pl.multiple_of(s * tp, 8)
    pltpu.sync_copy(stage_v, dega_sh.at[pl.ds(off, tp)])
    pltpu.sync_copy(stage_v, degb_sh.at[pl.ds(off, tp)])
    plsc.subcore_barrier()

    def body(j, carry):
      @pl.when(j < t)
      def _():
        pltpu.async_copy(ones_v, dega_sh.at[edge_v.at[j, 0]], sem, add=True)
        pltpu.async_copy(ones_v, degb_sh.at[edge_v.at[j, 1]], sem,
                         add=True)

      return carry

    lax.fori_loop(0, maxcpt, body, 0)

    def drain(j, carry):
      pltpu.make_async_copy(ones_v, dega_sh.at[edge_v.at[0, 0]], sem).wait()
      return carry

    lax.fori_loop(0, 2 * t, drain, 0)
    plsc.subcore_barrier()
    base = c * 2 * Np
    pltpu.sync_copy(dega_sh.at[pl.ds(off, tp)], stage_v)
    pltpu.sync_copy(stage_v,
                    out_hbm.at[pl.ds(pl.multiple_of(base + s * tp, 8), tp)])
    pltpu.sync_copy(degb_sh.at[pl.ds(off, tp)], stage_v)
    pltpu.sync_copy(stage_v,
                    out_hbm.at[pl.ds(pl.multiple_of(base + Np + s * tp, 8),
                                     tp)])

  return deg_kernel


def _make_prop_kernel(Np, H, nch):
  mesh = plsc.VectorSubcoreMesh(core_axis_name="c", subcore_axis_name="s")
  maxcpt = -(-nch // NW)
  rpt = Np // NS
  nq = rpt // CHUNK
  assert rpt % CHUNK == 0
  NSLOT = 6
  PD = 4

  @functools.partial(
      pl.kernel,
      out_type=jax.ShapeDtypeStruct((NC * NS * nq, 2, CHUNK, H), jnp.float32),
      mesh=mesh,
      compiler_params=pltpu.CompilerParams(use_tc_tiling_on_sc=False),
      scratch_types=[
          pltpu.VMEM((maxcpt, 2, CHUNK), jnp.int32),
          pltpu.VMEM((NSLOT, CHUNK, H), jnp.float32),
          pltpu.VMEM_SHARED((Np, H), jnp.float32),
      ] + [pltpu.SemaphoreType.DMA] * (2 * NSLOT),
  )
  def prop_kernel(table_hbm, edge_hbm, out_hbm,
                  edge_v, rows_v, acc_sh, *sems):
    c = lax.axis_index("c")
    s = lax.axis_index("s")
    w = c * NS + s
    cs = (nch * w) // NW
    ce = (nch * (w + 1)) // NW
    t = ce - cs
    sgs = sems[:NSLOT]
    sss = sems[NSLOT:]
    pltpu.sync_copy(edge_hbm.at[pl.ds(cs, maxcpt)], edge_v)

    def dbl(j, carry):
      for k in range(CHUNK // 16):
        v = edge_v[j, 0, pl.ds(k * 16, 16)]
        edge_v[j, 0, pl.ds(k * 16, 16)] = v + v
      return carry

    lax.fori_loop(0, maxcpt, dbl, 0)

    def zbody(j, carry):
      for k in range(H // 16):
        rows_v[0, j, pl.ds(k * 16, 16)] = jnp.zeros((16,), jnp.float32)
      return carry

    lax.fori_loop(0, CHUNK, zbody, 0)
    for q in range(nq):
      pltpu.sync_copy(rows_v.at[0],
                      acc_sh.at[pl.ds(pl.multiple_of(s * rpt + q * CHUNK, 8),
                                      CHUNK)])
    plsc.subcore_barrier()

    for b in range(PD):
      @pl.when(b < t)
      def _(b=b):
        pltpu.async_copy(table_hbm.at[edge_v.at[b, 0]], rows_v.at[b], sgs[b])

    def body(jo, carry):
      for b in range(NSLOT):
        jj = NSLOT * jo + b
        pj = jj + PD
        pb = (b + PD) % NSLOT

        @pl.when(pj < t)
        def _(jj=jj, pj=pj, pb=pb):
          @pl.when(pj >= NSLOT)
          def _():
            pltpu.make_async_copy(rows_v.at[pb],
                                  acc_sh.at[edge_v.at[pj - NSLOT, 1]],
                                  sss[pb]).wait()

          pltpu.async_copy(table_hbm.at[edge_v.at[pj, 0]], rows_v.at[pb],
                           sgs[pb])

        @pl.when(jj < t)
        def _(jj=jj, b=b):
          pltpu.make_async_copy(table_hbm.at[edge_v.at[jj, 0]], rows_v.at[b],
                                sgs[b]).wait()
          pltpu.async_copy(rows_v.at[b], acc_sh.at[edge_v.at[jj, 1]],
                           sss[b], add=True)

      return carry

    _ = lax.fori_loop(0, (maxcpt + NSLOT - 1) // NSLOT, body, 0)

    for b in range(NSLOT):
      @pl.when(b < t)
      def _(b=b):
        last = ((t - 1 - b) // NSLOT) * NSLOT + b
        pltpu.make_async_copy(rows_v.at[b],
                              acc_sh.at[edge_v.at[last, 1]],
                              sss[b]).wait()

    plsc.subcore_barrier()

    for q in range(3):
      if q < nq:
        p = (q % 3) * 2
        aq = pl.multiple_of(s * rpt + q * CHUNK, 8)
        pltpu.async_copy(acc_sh.at[pl.ds(aq, CHUNK)], rows_v.at[p], sgs[p])
    for q in range(nq):
      p = (q % 3) * 2
      rq = q + 1
      if q >= 2 and rq < nq:
        rp = (rq % 3) * 2
        roi = (c * NS + s) * nq + (rq - 3)
        pltpu.make_async_copy(rows_v.at[pl.ds(rp, 2)], out_hbm.at[roi],
                              sss[rp]).wait()
        aq2 = pl.multiple_of(s * rpt + rq * CHUNK, 8)
        pltpu.async_copy(acc_sh.at[pl.ds(aq2, CHUNK)], rows_v.at[rp], sgs[rp])
      aq = pl.multiple_of(s * rpt + q * CHUNK, 8)
      pltpu.make_async_copy(acc_sh.at[pl.ds(aq, CHUNK)], rows_v.at[p],
                            sgs[p]).wait()

      def spread_hi(i, carry, p=p):
        for u in range(4):
          j = 64 + 4 * i + u
          for k in range(H // 16):
            rows_v[p + 1, 2 * j - 128, pl.ds(k * 16, 16)] = (
                rows_v[p, j, pl.ds(k * 16, 16)])
        return carry

      lax.fori_loop(0, 16, spread_hi, 0)

      def spread_lo(i, carry, p=p):
        for u in range(3):
          j = 63 - (3 * i + u)
          for k in range(H // 16):
            rows_v[p, 2 * j, pl.ds(k * 16, 16)] = (
                rows_v[p, j, pl.ds(k * 16, 16)])
        return carry

      lax.fori_loop(0, 21, spread_lo, 0)

      oi = (c * NS + s) * nq + q
      pltpu.async_copy(rows_v.at[pl.ds(p, 2)], out_hbm.at[oi], sss[p])
    for q in range(max(nq - 3, 0), nq):
      p = (q % 3) * 2
      oi = (c * NS + s) * nq + q
      pltpu.make_async_copy(rows_v.at[pl.ds(p, 2)], out_hbm.at[oi],
                            sss[p]).wait()

  return prop_kernel


def _norms(deg_ref, Np, blk, i, which):
  d0 = deg_ref[pl.ds(which * Np + i * blk, blk)]
  d1 = deg_ref[pl.ds((2 + which) * Np + i * blk, blk)]
  n = lax.rsqrt(jnp.maximum(d0 + d1, 1.0))
  return jnp.reshape(n, (blk, 1))


def _prep_body(Np, blk, x_ref, w1_ref, deg_ref, g1_ref):
  i = pl.program_id(0)
  nsrc = _norms(deg_ref, Np, blk, i, 0)
  h = jnp.dot(x_ref[...], w1_ref[...], preferred_element_type=jnp.float32)
  g = h * nsrc
  g1_ref[...] = jnp.concatenate([g, jnp.zeros_like(g)], axis=1)


def _mid_body(Np, blk, p_ref, deg_ref, o_ref):
  i = pl.program_id(0)
  nsrc = _norms(deg_ref, Np, blk, i, 0)
  ndst = _norms(deg_ref, Np, blk, i, 1)
  pv = p_ref[...]
  H = pv.shape[-1] // 2
  agg = pv[0, :, :H] + pv[1, :, :H]
  h1 = jnp.maximum(agg * ndst, 0.0)
  h = h1 * nsrc
  o_ref[...] = jnp.concatenate([h, jnp.zeros_like(h)], axis=1)


def _dec_body(Np, blk, q_ref, deg_ref, w2_ref, w3_ref, mu_ref, lv_ref):
  i = pl.program_id(0)
  ndst = _norms(deg_ref, Np, blk, i, 1)
  qv = q_ref[...]
  H = qv.shape[-1] // 2
  sblk = (qv[0, :, :H] + qv[1, :, :H]) * ndst
  mu_ref[...] = jnp.dot(sblk, w2_ref[...], preferred_element_type=jnp.float32)
  lv_ref[...] = jnp.dot(sblk, w3_ref[...], preferred_element_type=jnp.float32)


def _adj_body(a_ref, b_ref, o_ref):
  o_ref[...] = lax.dot_general(a_ref[...], b_ref[...],
                               (((1,), (1,)), ((), ())),
                               preferred_element_type=jnp.float32)


def kernel(x, edge_index, W1, W2, W3):
  N, D = x.shape
  E = edge_index.shape[1]
  H1 = W1.shape[1]
  H2 = W2.shape[1]

  Np = _round_up(N, 2048)
  blk = 2048
  nblk = Np // blk
  nch = E // CHUNK
  assert E % CHUNK == 0

  edge_r = jnp.transpose(edge_index.reshape(2, nch, CHUNK), (1, 0, 2))

  deg_flat = _make_deg_kernel(Np, nch)(edge_r)

  x_pad = jnp.pad(x, ((0, Np - N), (0, 0)))
  g1 = pl.pallas_call(
      functools.partial(_prep_body, Np, blk),
      grid=(nblk,),
      in_specs=[
          pl.BlockSpec((blk, D), lambda i: (i, 0)),
          pl.BlockSpec((D, H1), lambda i: (0, 0)),
          pl.BlockSpec((NC * 2 * Np,), lambda i: (0,)),
      ],
      out_specs=pl.BlockSpec((blk, 2 * H1), lambda i: (i, 0)),
      out_shape=jax.ShapeDtypeStruct((Np, 2 * H1), jnp.float32),
  )(x_pad, W1, deg_flat)
  g1t = g1.reshape(2 * Np, H1)

  prop = _make_prop_kernel(Np, H1, nch)
  agg_parts = prop(g1t, edge_r).reshape(NC, Np, 2 * H1)

  h1n = pl.pallas_call(
      functools.partial(_mid_body, Np, blk),
      grid=(nblk,),
      in_specs=[
          pl.BlockSpec((NC, blk, 2 * H1), lambda i: (0, i, 0)),
          pl.BlockSpec((NC * 2 * Np,), lambda i: (0,)),
      ],
      out_specs=pl.BlockSpec((blk, 2 * H1), lambda i: (i, 0)),
      out_shape=jax.ShapeDtypeStruct((Np, 2 * H1), jnp.float32),
  )(agg_parts, deg_flat)
  h1nt = h1n.reshape(2 * Np, H1)

  q_parts = prop(h1nt, edge_r).reshape(NC, Np, 2 * H1)

  mu, logvar = pl.pallas_call(
      functools.partial(_dec_body, Np, blk),
      grid=(nblk,),
      in_specs=[
          pl.BlockSpec((NC, blk, 2 * H1), lambda i: (0, i, 0)),
          pl.BlockSpec((NC * 2 * Np,), lambda i: (0,)),
          pl.BlockSpec((H1, H2), lambda i: (0, 0)),
          pl.BlockSpec((H1, H2), lambda i: (0, 0)),
      ],
      out_specs=[
          pl.BlockSpec((blk, H2), lambda i: (i, 0)),
          pl.BlockSpec((blk, H2), lambda i: (i, 0)),
      ],
      out_shape=[
          jax.ShapeDtypeStruct((N, H2), jnp.float32),
          jax.ShapeDtypeStruct((N, H2), jnp.float32),
      ],
  )(q_parts, deg_flat, W2, W3)

  bm = 400
  adj = pl.pallas_call(
      _adj_body,
      grid=(N // bm,),
      in_specs=[
          pl.BlockSpec((bm, H2), lambda i: (i, 0)),
          pl.BlockSpec((N, H2), lambda i: (0, 0)),
      ],
      out_specs=pl.BlockSpec((bm, N), lambda i: (i, 0)),
      out_shape=jax.ShapeDtypeStruct((N, N), jnp.float32),
  )(mu, mu)

  return (adj, mu, logvar)
